# baseline (speedup 1.0000x reference)
import functools

import jax
import jax.numpy as jnp
from jax import lax
from jax.experimental import pallas as pl
from jax.experimental.pallas import tpu as pltpu
from jax.experimental.pallas import tpu_sc as plsc

NC, NS, LANES = 2, 16, 16

C_G = 640
C_S = 640
CHUNK = 1536
CPAD = CHUNK + 16
N_CHUNKS_PER_CORE = 33
ZROWS = 48
FLUSH = 256


def _cdiv(a, b):
    return -(-a // b)


def _ceil_to(x, m):
    return _cdiv(x, m) * m


def _mesh():
    return plsc.VectorSubcoreMesh(
        core_axis_name="c", subcore_axis_name="s",
        num_cores=NC, num_subcores=NS)


def _sc_gather(h_g1, idx_list):
    n1, u = h_g1.shape
    fs = [int(i.shape[0]) for i in idx_list]
    ws = [_ceil_to(_cdiv(f, NC * NS), 8) for f in fs]
    out_type = tuple(jax.ShapeDtypeStruct((f, u), jnp.float32) for f in fs)

    @functools.partial(
        pl.kernel, out_type=out_type, mesh=_mesh(),
        scratch_types=(
            pltpu.VMEM((C_G,), jnp.int32),
            pltpu.VMEM((C_G, u), jnp.float32),
            pltpu.SemaphoreType.DMA,
        ))
    def body(h_ref, i2, i3, i4, o2, o3, o4, idx_v, rows_v, sem):
        wid = lax.axis_index("s") * NC + lax.axis_index("c")
        for idx_ref, out_ref, f, w in zip((i2, i3, i4), (o2, o3, o4), fs, ws):
            base = wid * w
            vw = jnp.minimum(w, f - base)
            nck = _cdiv(w, C_G)

            @pl.loop(0, nck)
            def _chunk(c):
                s = base + jnp.minimum(c * C_G, vw - C_G)
                pltpu.sync_copy(idx_ref.at[pl.ds(s, C_G)], idx_v)
                pltpu.async_copy(h_ref.at[idx_v], rows_v, sem).wait()
                pltpu.sync_copy(rows_v, out_ref.at[pl.ds(s, C_G)])

    return body(h_g1, *idx_list)


def _tc_gru(y_prev, obase, m_flat, t_steps, ng, w_iht, w_hht, b_ih2, b_hh2,
            w1kt, ftot):
    u = m_flat.shape[1]
    r = 1000
    nblk = ng // r

    def body(*refs):
        if y_prev is None:
            x_ref, wih, whh, bih, bhh, w1k, out, h_ref = refs
        else:
            _, x_ref, wih, whh, bih, bhh, w1k, out, h_ref = refs
        t = pl.program_id(1)

        @pl.when(t == 0)
        def _init():
            h_ref[...] = jnp.zeros((r, u), jnp.float32)

        x = x_ref[...]
        h = h_ref[...]
        gi = jnp.dot(x, wih[...], preferred_element_type=jnp.float32) + bih[...]
        gh = jnp.dot(h, whh[...], preferred_element_type=jnp.float32) + bhh[...]
        rg = jax.nn.sigmoid(gi[:, :u] + gh[:, :u])
        zg = jax.nn.sigmoid(gi[:, u:2 * u] + gh[:, u:2 * u])
        nn = jnp.tanh(gi[:, 2 * u:] + rg * gh[:, 2 * u:])
        h = (1.0 - zg) * nn + zg * h
        h_ref[...] = h
        out[...] = jnp.dot(h, w1k[...], preferred_element_type=jnp.float32)

    x_spec = pl.BlockSpec((r, u), lambda i, t: (t * nblk + i, 0))
    w_specs = [
        pl.BlockSpec((u, 3 * u), lambda i, t: (0, 0)),
        pl.BlockSpec((u, 3 * u), lambda i, t: (0, 0)),
        pl.BlockSpec((1, 3 * u), lambda i, t: (0, 0)),
        pl.BlockSpec((1, 3 * u), lambda i, t: (0, 0)),
        pl.BlockSpec((u, u), lambda i, t: (0, 0)),
    ]
    ob = obase // r
    out_spec = pl.BlockSpec((r, u), lambda i, t: (ob + t * nblk + i, 0))
    in_specs = [x_spec] + w_specs
    args = [m_flat, w_iht, w_hht, b_ih2, b_hh2, w1kt]
    aliases = {}
    if y_prev is not None:
        in_specs = [pl.BlockSpec(memory_space=pl.ANY)] + in_specs
        args = [y_prev] + args
        aliases = {0: 0}
    return pl.pallas_call(
        body,
        grid=(nblk, t_steps),
        in_specs=in_specs,
        out_specs=out_spec,
        out_shape=jax.ShapeDtypeStruct((ftot, u), jnp.float32),
        scratch_shapes=[pltpu.VMEM((r, u), jnp.float32)],
        input_output_aliases=aliases,
    )(*args)


def _tc_counts(idx2d, n_buckets):
    ns, w = idx2d.shape

    def body(i_ref, out):
        bucket = i_ref[...] // CHUNK
        cols = [jnp.sum((bucket == b).astype(jnp.float32), axis=1)
                for b in range(n_buckets)]
        cols = cols + [cols[0] * 0.0] * (128 - n_buckets)
        out[...] = jnp.stack(cols, axis=1)

    return pl.pallas_call(
        body,
        out_shape=jax.ShapeDtypeStruct((ns, 128), jnp.float32),
    )(idx2d)


def _sc_scatter(y_all, idx_pad, counts_rows, u, ftot):
    w16 = idx_pad.shape[0] // NS
    nck = _cdiv(w16, C_S)
    slots = nck * C_S
    rsz = w16 + LANES
    rows_per_sub = CHUNK // NS
    nb = NC * N_CHUNKS_PER_CORE
    nbp = _ceil_to(nb, 8)

    @functools.partial(
        pl.kernel,
        out_type=jax.ShapeDtypeStruct((nb * CHUNK, u), jnp.float32),
        mesh=_mesh(),
        scratch_types=(
            pltpu.VMEM((w16,), jnp.int32),
            pltpu.VMEM((1, C_S), jnp.int32),
            pltpu.VMEM((C_S,), jnp.int32),
            pltpu.VMEM((C_S,), jnp.int32),
            pltpu.VMEM((FLUSH,), jnp.int32),
            pltpu.VMEM((FLUSH,), jnp.int32),
            pltpu.VMEM((1, FLUSH), jnp.int32),
            pltpu.VMEM((FLUSH, u), jnp.float32),
            pltpu.VMEM((ZROWS, u), jnp.float32),
            pltpu.VMEM((nbp, 128), jnp.float32),
            pltpu.VMEM((64,), jnp.int32),
            pltpu.VMEM((2, LANES), jnp.int32),
            pltpu.VMEM((FLUSH,), jnp.int32),
            pltpu.VMEM_SHARED((CPAD, u), jnp.float32),
            pltpu.VMEM_SHARED((NS * rsz,), jnp.int32),
            pltpu.SemaphoreType.DMA,
        ))
    def body(y, idx, cnt_hbm, out, idxbuf, posb, encs, idxr, encw, gidx,
             l2d, pay, zbuf, cnts, tmp, tcur, zero256, acc, encsh, sem):
        cid = lax.axis_index("c")
        sid = lax.axis_index("s")
        wid = sid * NC + cid

        @pl.loop(0, ZROWS)
        def _zrow(zr):
            for j in range(u // LANES):
                zbuf[zr, pl.ds(j * LANES, LANES)] = jnp.zeros(
                    (LANES,), jnp.float32)

        pltpu.sync_copy(cnt_hbm.at[pl.ds(sid * nbp, nbp)], cnts)
        for j in range(4):
            tmp[pl.ds(j * LANES, LANES)] = jnp.zeros((LANES,), jnp.int32)
        for j in range(FLUSH // LANES):
            zero256[pl.ds(j * LANES, LANES)] = jnp.zeros((LANES,), jnp.int32)

        vw = jnp.minimum(w16, ftot - sid * w16)
        rbase = sid * rsz

        @pl.loop(0, N_CHUNKS_PER_CORE)
        def _per_chunk(ci):
            lo = (cid * N_CHUNKS_PER_CORE + ci) * CHUNK

            @pl.loop(0, rows_per_sub // ZROWS)
            def _zero(z):
                pltpu.sync_copy(
                    zbuf, acc.at[pl.ds(sid * rows_per_sub + z * ZROWS, ZROWS)])

            @pl.when(sid == 0)
            def _zero_tail():
                pltpu.sync_copy(zbuf.at[pl.ds(0, CPAD - CHUNK)],
                                acc.at[pl.ds(CHUNK, CPAD - CHUNK)])

            plsc.subcore_barrier()

            cvec = cnts[cid * N_CHUNKS_PER_CORE + ci, pl.ds(0, LANES)]
            cur = jnp.int32(cvec[0])
            trips = (cur + (FLUSH - 1)) // FLUSH

            @pl.loop(0, trips)
            def _zenc(f):
                pltpu.sync_copy(zero256,
                                encsh.at[pl.ds(rbase + f * FLUSH, FLUSH)])

            tcur[1, pl.ds(0, LANES)] = jnp.zeros((LANES,), jnp.int32)

            @pl.loop(0, nck)
            def _scan(c):
                sl = jnp.minimum(c * C_S, vw - C_S)
                off = sl - c * C_S
                pltpu.sync_copy(idx.at[pl.ds(sid * w16 + sl, C_S)], idxr)
                iota = lax.iota(jnp.int32, LANES)
                zero = jnp.zeros((LANES,), jnp.int32)
                one = jnp.full((LANES,), 1, jnp.int32)
                dumpv = iota + w16
                for j in range(C_S // LANES):
                    v = idxr[pl.ds(j * LANES, LANES)]
                    posrel = iota + (off + j * LANES)
                    local = v - lo
                    ok = (posrel >= 0) & (local >= 0) & (local < CHUNK)
                    oki = jnp.where(ok, one, zero)
                    tmp[pl.ds(LANES, LANES)] = oki
                    pref = oki
                    for d in (1, 2, 4, 8):
                        pref = pref + tmp[pl.ds(LANES - d, LANES)]
                        tmp[pl.ds(LANES, LANES)] = pref
                    curv = tcur[1, pl.ds(0, LANES)]
                    pos = jnp.where(ok, curv + (pref - 1), dumpv)
                    pos = jnp.minimum(jnp.maximum(pos, 0),
                                      rsz - 1) + rbase
                    gp = iota + ((sid * w16) + sl + j * LANES)
                    enc = gp * 4096 + jnp.where(ok, local, zero)
                    posb[0, pl.ds(j * LANES, LANES)] = pos
                    encs[pl.ds(j * LANES, LANES)] = enc
                    tcur[1, pl.ds(0, LANES)] = curv + pref[LANES - 1]
                pltpu.sync_copy(encs, encsh.at[posb.at[0]], add=True)

            @pl.loop(0, trips)
            def _flush(f):
                pltpu.sync_copy(encsh.at[pl.ds(rbase + f * FLUSH, FLUSH)],
                                encw)
                iota = lax.iota(jnp.int32, LANES)
                zero = jnp.zeros((LANES,), jnp.int32)
                garb = jnp.full((LANES,), CHUNK, jnp.int32)
                for q in range(FLUSH // LANES):
                    e = encw[pl.ds(q * LANES, LANES)]
                    okw = (iota + (f * FLUSH + q * LANES)) < cur
                    gp = jnp.where(okw, jnp.right_shift(e, 12), zero)
                    gidx[pl.ds(q * LANES, LANES)] = jnp.minimum(
                        jnp.maximum(gp, 0), ftot - 1)
                    ld = jnp.where(okw, jnp.bitwise_and(e, 4095), garb)
                    l2d[0, pl.ds(q * LANES, LANES)] = jnp.minimum(ld, CHUNK)
                pltpu.async_copy(y.at[gidx], pay, sem).wait()
                pltpu.sync_copy(pay, acc.at[l2d.at[0]], add=True)

            plsc.subcore_barrier()

            pltpu.sync_copy(
                acc.at[pl.ds(sid * rows_per_sub, rows_per_sub)],
                out.at[pl.ds(lo + sid * rows_per_sub, rows_per_sub)])

            plsc.subcore_barrier()

    return body(y_all, idx_pad, counts_rows)


def _tc_dense(h_g1, scat, w1at, b1r, w2t, b2r):
    n1, u = h_g1.shape
    r = 1000
    nblk = _cdiv(n1, r)

    def body(h_ref, s_ref, w1a, b1_, w2, b2_, out):
        pre = (jnp.dot(h_ref[...], w1a[...],
                       preferred_element_type=jnp.float32)
               + s_ref[...] + b1_[...])
        hid = jnp.tanh(pre)
        out[...] = (jnp.dot(hid, w2[...], preferred_element_type=jnp.float32)
                    + b2_[...])

    return pl.pallas_call(
        body,
        grid=(nblk,),
        in_specs=[
            pl.BlockSpec((r, u), lambda i: (i, 0)),
            pl.BlockSpec((r, u), lambda i: (i, 0)),
            pl.BlockSpec((u, u), lambda i: (0, 0)),
            pl.BlockSpec((1, u), lambda i: (0, 0)),
            pl.BlockSpec((u, u), lambda i: (0, 0)),
            pl.BlockSpec((1, u), lambda i: (0, 0)),
        ],
        out_specs=pl.BlockSpec((r, u), lambda i: (i, 0)),
        out_shape=jax.ShapeDtypeStruct((n1, u), jnp.float32),
    )(h_g1, scat, w1at, b1r, w2t, b2r)


def kernel(h_g1, member2, member3, member4, W_ih, W_hh, b_ih, b_hh, W1, b1,
           W2, b2):
    n1, u = h_g1.shape
    ng = member2.shape[0]

    idx_list = [m.T.reshape(-1) for m in (member2, member3, member4)]
    cat_idx = jnp.concatenate(idx_list)
    ftot = int(cat_idx.shape[0])

    w_iht = W_ih.T
    w_hht = W_hh.T
    b_ih2 = b_ih.reshape(1, 3 * u)
    b_hh2 = b_hh.reshape(1, 3 * u)
    w1t = W1.T
    w2t = W2.T

    m_list = _sc_gather(h_g1, idx_list)

    y_all = None
    obase = 0
    for k, (m_flat, member) in enumerate(zip(m_list,
                                             (member2, member3, member4))):
        t_steps = member.shape[1]
        w1kt = w1t[(k + 1) * u:(k + 2) * u, :]
        y_all = _tc_gru(y_all, obase, m_flat, t_steps, ng, w_iht, w_hht,
                        b_ih2, b_hh2, w1kt, ftot)
        obase += t_steps * ng

    w16 = _ceil_to(_cdiv(ftot, NS), 8)
    nb = NC * N_CHUNKS_PER_CORE
    idx_pad = jnp.pad(cat_idx, (0, NS * w16 - ftot),
                      constant_values=jnp.int32(1 << 28))
    counts = _tc_counts(idx_pad.reshape(NS, w16), nb)
    nbp = _ceil_to(nb, 8)
    counts_p = jnp.pad(counts[:, :nb], ((0, 0), (0, nbp - nb)))
    counts_rows = jnp.broadcast_to(
        counts_p.reshape(NS * nbp, 1), (NS * nbp, 128))
    scat = _sc_scatter(y_all, idx_pad, counts_rows, u, ftot)

    return _tc_dense(h_g1, scat, w1t[:u, :], b1.reshape(1, u), w2t,
                     b2.reshape(1, u))


# speedup vs baseline: 1.1841x; 1.1841x over previous
import functools

import jax
import jax.numpy as jnp
from jax import lax
from jax.experimental import pallas as pl
from jax.experimental.pallas import tpu as pltpu
from jax.experimental.pallas import tpu_sc as plsc

NC, NS, LANES = 2, 16, 16

C_G = 640
C_S = 640
CHUNK = 3328
CPAD = CHUNK + 16
N_CHUNKS_PER_CORE = 16
ZROWS = 104


def _cdiv(a, b):
    return -(-a // b)


def _ceil_to(x, m):
    return _cdiv(x, m) * m


def _mesh():
    return plsc.VectorSubcoreMesh(
        core_axis_name="c", subcore_axis_name="s",
        num_cores=NC, num_subcores=NS)


def _sc_gather(h_g1, idx_list):
    n1, u = h_g1.shape
    fs = [int(i.shape[0]) for i in idx_list]
    ws = [_ceil_to(_cdiv(f, NC * NS), 8) for f in fs]
    out_type = tuple(jax.ShapeDtypeStruct((f, u), jnp.float32) for f in fs)

    @functools.partial(
        pl.kernel, out_type=out_type, mesh=_mesh(),
        scratch_types=(
            pltpu.VMEM((C_G,), jnp.int32),
            pltpu.VMEM((C_G, u), jnp.float32),
            pltpu.SemaphoreType.DMA,
        ))
    def body(h_ref, i2, i3, i4, o2, o3, o4, idx_v, rows_v, sem):
        wid = lax.axis_index("s") * NC + lax.axis_index("c")
        for idx_ref, out_ref, f, w in zip((i2, i3, i4), (o2, o3, o4), fs, ws):
            base = wid * w
            vw = jnp.minimum(w, f - base)
            nck = _cdiv(w, C_G)

            @pl.loop(0, nck)
            def _chunk(c):
                s = base + jnp.minimum(c * C_G, vw - C_G)
                pltpu.sync_copy(idx_ref.at[pl.ds(s, C_G)], idx_v)
                pltpu.async_copy(h_ref.at[idx_v], rows_v, sem).wait()
                pltpu.sync_copy(rows_v, out_ref.at[pl.ds(s, C_G)])

    return body(h_g1, *idx_list)


def _tc_gru(m_flat, t_steps, ng, w_iht, w_hht, b_ih2, b_hh2, w1kt):
    u = m_flat.shape[1]
    r = 1000
    nblk = ng // r

    def body(*refs):
        xs = refs[:t_steps]
        wih, whh, bih, bhh, w1k, out = refs[t_steps:]
        h = jnp.zeros((r, u), jnp.float32)
        for t in range(t_steps):
            x = xs[t][...]
            gi = jnp.dot(x, wih[...], preferred_element_type=jnp.float32) + bih[...]
            gh = jnp.dot(h, whh[...], preferred_element_type=jnp.float32) + bhh[...]
            rg = jax.nn.sigmoid(gi[:, :u] + gh[:, :u])
            zg = jax.nn.sigmoid(gi[:, u:2 * u] + gh[:, u:2 * u])
            ng_ = jnp.tanh(gi[:, 2 * u:] + rg * gh[:, 2 * u:])
            h = (1.0 - zg) * ng_ + zg * h
            out[t] = jnp.dot(h, w1k[...], preferred_element_type=jnp.float32)

    x_specs = [
        pl.BlockSpec((r, u), lambda i, t=t: (t * nblk + i, 0))
        for t in range(t_steps)
    ]
    w_specs = [
        pl.BlockSpec((u, 3 * u), lambda i: (0, 0)),
        pl.BlockSpec((u, 3 * u), lambda i: (0, 0)),
        pl.BlockSpec((1, 3 * u), lambda i: (0, 0)),
        pl.BlockSpec((1, 3 * u), lambda i: (0, 0)),
        pl.BlockSpec((u, u), lambda i: (0, 0)),
    ]
    return pl.pallas_call(
        body,
        grid=(nblk,),
        in_specs=x_specs + w_specs,
        out_specs=pl.BlockSpec((t_steps, r, u), lambda i: (0, i, 0)),
        out_shape=jax.ShapeDtypeStruct((t_steps, ng, u), jnp.float32),
    )(*([m_flat] * t_steps), w_iht, w_hht, b_ih2, b_hh2, w1kt)


def _sc_scatter(y_list, idx_list, n1, u):
    fs = [int(i.shape[0]) for i in idx_list]
    ws = [_ceil_to(_cdiv(f, NS), 8) for f in fs]
    ncks = [_cdiv(w, C_S) for w in ws]
    rows_per_sub = CHUNK // NS

    @functools.partial(
        pl.kernel,
        out_type=jax.ShapeDtypeStruct((NC * N_CHUNKS_PER_CORE * CHUNK, u),
                                      jnp.float32),
        mesh=_mesh(),
        scratch_types=(
            pltpu.VMEM((C_S,), jnp.int32),
            pltpu.VMEM((C_S,), jnp.int32),
            pltpu.VMEM((C_S, u), jnp.float32),
            pltpu.VMEM((ZROWS, u), jnp.float32),
            pltpu.VMEM_SHARED((CPAD, u), jnp.float32),
            pltpu.SemaphoreType.DMA,
        ))
    def body(y2, y3, y4, i2, i3, i4, out, idxr, idxl, pay, zbuf, acc, sem):
        cid = lax.axis_index("c")
        sid = lax.axis_index("s")

        @pl.loop(0, ZROWS)
        def _zrow(zr):
            for j in range(u // LANES):
                zbuf[zr, pl.ds(j * LANES, LANES)] = jnp.zeros(
                    (LANES,), jnp.float32)

        @pl.loop(0, N_CHUNKS_PER_CORE)
        def _per_chunk(ci):
            lo = (cid * N_CHUNKS_PER_CORE + ci) * CHUNK

            @pl.loop(0, rows_per_sub // ZROWS)
            def _zero(z):
                pltpu.sync_copy(
                    zbuf, acc.at[pl.ds(sid * rows_per_sub + z * ZROWS, ZROWS)])

            @pl.when(sid == 0)
            def _zero_tail():
                pltpu.sync_copy(zbuf.at[pl.ds(0, CPAD - CHUNK)],
                                acc.at[pl.ds(CHUNK, CPAD - CHUNK)])

            plsc.subcore_barrier()

            for y_ref, idx_ref, f, w, nck in zip(
                    (y2, y3, y4), (i2, i3, i4), fs, ws, ncks):
                base = sid * w
                vw = jnp.minimum(w, f - base)

                @pl.loop(0, nck)
                def _chunk(c):
                    s = base + jnp.minimum(c * C_S, vw - C_S)
                    pltpu.sync_copy(idx_ref.at[pl.ds(s, C_S)], idxr)
                    pltpu.sync_copy(y_ref.at[pl.ds(s, C_S)], pay)
                    off = s - (base + c * C_S)
                    for j in range(C_S // LANES):
                        v = idxr[pl.ds(j * LANES, LANES)]
                        posrel = (jnp.arange(LANES, dtype=jnp.int32)
                                  + (off + j * LANES))
                        local = v - lo
                        ok = ((posrel >= 0) & (local >= 0) & (local < CHUNK))
                        idxl[pl.ds(j * LANES, LANES)] = jnp.where(
                            ok, local, jnp.full((LANES,), CHUNK, jnp.int32))
                    pltpu.sync_copy(pay, acc.at[idxl], add=True)

            plsc.subcore_barrier()

            pltpu.sync_copy(
                acc.at[pl.ds(sid * rows_per_sub, rows_per_sub)],
                out.at[pl.ds(lo + sid * rows_per_sub, rows_per_sub)])

            plsc.subcore_barrier()

    return body(*y_list, *idx_list)


def _tc_dense(h_g1, scat, w1at, b1r, w2t, b2r):
    n1, u = h_g1.shape
    r = 1000
    nblk = _cdiv(n1, r)

    def body(h_ref, s_ref, w1a, b1_, w2, b2_, out):
        pre = (jnp.dot(h_ref[...], w1a[...],
                       preferred_element_type=jnp.float32)
               + s_ref[...] + b1_[...])
        hid = jnp.tanh(pre)
        out[...] = (jnp.dot(hid, w2[...], preferred_element_type=jnp.float32)
                    + b2_[...])

    return pl.pallas_call(
        body,
        grid=(nblk,),
        in_specs=[
            pl.BlockSpec((r, u), lambda i: (i, 0)),
            pl.BlockSpec((r, u), lambda i: (i, 0)),
            pl.BlockSpec((u, u), lambda i: (0, 0)),
            pl.BlockSpec((1, u), lambda i: (0, 0)),
            pl.BlockSpec((u, u), lambda i: (0, 0)),
            pl.BlockSpec((1, u), lambda i: (0, 0)),
        ],
        out_specs=pl.BlockSpec((r, u), lambda i: (i, 0)),
        out_shape=jax.ShapeDtypeStruct((n1, u), jnp.float32),
    )(h_g1, scat, w1at, b1r, w2t, b2r)


def kernel(h_g1, member2, member3, member4, W_ih, W_hh, b_ih, b_hh, W1, b1,
           W2, b2):
    n1, u = h_g1.shape
    ng = member2.shape[0]

    idx_list = [m.T.reshape(-1) for m in (member2, member3, member4)]

    w_iht = W_ih.T
    w_hht = W_hh.T
    b_ih2 = b_ih.reshape(1, 3 * u)
    b_hh2 = b_hh.reshape(1, 3 * u)
    w1t = W1.T
    w2t = W2.T

    m2, m3, m4 = _sc_gather(h_g1, idx_list)

    ys = []
    for k, (m_flat, member) in enumerate(zip((m2, m3, m4),
                                             (member2, member3, member4))):
        t_steps = member.shape[1]
        w1kt = w1t[(k + 1) * u:(k + 2) * u, :]
        y = _tc_gru(m_flat, t_steps, ng, w_iht, w_hht, b_ih2, b_hh2, w1kt)
        ys.append(y.reshape(t_steps * ng, u))

    scat = _sc_scatter(ys, idx_list, n1, u)

    return _tc_dense(h_g1, scat, w1t[:u, :], b1.reshape(1, u), w2t,
                     b2.reshape(1, u))
